# baseline (speedup 1.0000x reference)
import functools

import jax
import jax.numpy as jnp
from jax import lax
from jax.experimental import pallas as pl
from jax.experimental.pallas import tpu as pltpu
from jax.experimental.pallas import tpu_sc as plsc

_B = 4
_S = 2048
_D = 512
_NW = 32
_C = _S // _NW
_R = 8
_NC = _C // _R


def _emb_kernel(idx_hbm, word_hbm, pos_hbm, out_hbm,
                idx_v, pos_v, wbuf, si, sp, sg0, sg1, so0, so1):
    wid = lax.axis_index("s") * 2 + lax.axis_index("c")
    pos_base = wid * _C
    idx_cps = [pltpu.async_copy(idx_hbm.at[b, pl.ds(pos_base, _C)],
                                idx_v.at[b], si) for b in range(_B)]
    pp = pltpu.async_copy(pos_hbm.at[pl.ds(pos_base, _C)], pos_v, sp)
    for c in idx_cps:
        c.wait()

    sgs, sos = (sg0, sg1), (so0, so1)

    def gather(chunk, buf):
        return [pltpu.async_copy(
            word_hbm.at[idx_v.at[b, pl.ds(chunk * _R, _R)]],
            wbuf.at[buf, b], sgs[buf]) for b in range(_B)]

    gs = [gather(0, 0), None]
    outs = [None, None]
    pp.wait()
    for c in range(_NC):
        cur, nxt = c % 2, (c + 1) % 2
        if c + 1 < _NC:
            if outs[nxt] is not None:
                for o in outs[nxt]:
                    o.wait()
            gs[nxt] = gather(c + 1, nxt)
        for g in gs[cur]:
            g.wait()

        def add_body(r, _, c=c, cur=cur):
            for g in range(_D // 64):
                vals = [pos_v[c * _R + r, pl.ds((g * 4 + j) * 16, 16)]
                        for j in range(4)]
                for j in range(4):
                    for b in range(_B):
                        plsc.addupdate(
                            wbuf.at[cur, b, r, pl.ds((g * 4 + j) * 16, 16)],
                            vals[j])
            return 0

        lax.fori_loop(0, _R, add_body, 0)
        outs[cur] = [pltpu.async_copy(
            wbuf.at[cur, b],
            out_hbm.at[b, pl.ds(pos_base + c * _R, _R)],
            sos[cur]) for b in range(_B)]
    for os in outs:
        for o in os:
            o.wait()


@jax.jit
def _run(inputs, word_emb, pos_emb):
    mesh = plsc.VectorSubcoreMesh(core_axis_name="c", subcore_axis_name="s")
    k = functools.partial(
        pl.kernel,
        mesh=mesh,
        out_type=jax.ShapeDtypeStruct((_B, _S, _D), jnp.float32),
        scratch_types=[
            pltpu.VMEM((_B, _C), jnp.int32),
            pltpu.VMEM((_C, _D), jnp.float32),
            pltpu.VMEM((2, _B, _R, _D), jnp.float32),
            pltpu.SemaphoreType.DMA,
            pltpu.SemaphoreType.DMA,
            pltpu.SemaphoreType.DMA,
            pltpu.SemaphoreType.DMA,
            pltpu.SemaphoreType.DMA,
            pltpu.SemaphoreType.DMA,
        ],
    )(_emb_kernel)
    return k(inputs, word_emb, pos_emb)


def kernel(inputs, word_emb, pos_emb):
    return _run(inputs, word_emb, pos_emb)


# speedup vs baseline: 1.0995x; 1.0995x over previous
import functools

import jax
import jax.numpy as jnp
from jax import lax
from jax.experimental import pallas as pl
from jax.experimental.pallas import tpu as pltpu
from jax.experimental.pallas import tpu_sc as plsc

_B = 4
_S = 2048
_D = 512
_NW = 32
_C = _S // _NW
_R = 16
_NC = _C // _R
_NB = 3


def _emb_kernel(idx_hbm, word_hbm, pos_hbm, out_hbm,
                idx_v, pos_v, wbuf, si,
                sp0, sp1, sp2, sg0, sg1, sg2, so0, so1, so2):
    wid = lax.axis_index("s") * 2 + lax.axis_index("c")
    pos_base = wid * _C
    idx_cps = [pltpu.async_copy(idx_hbm.at[b, pl.ds(pos_base, _C)],
                                idx_v.at[b], si) for b in range(_B)]
    sps, sgs, sos = (sp0, sp1, sp2), (sg0, sg1, sg2), (so0, so1, so2)
    for cp in idx_cps:
        cp.wait()

    def pos_load(c):
        return pltpu.async_copy(
            pos_hbm.at[pl.ds(pos_base + c * _R, _R)],
            pos_v.at[c % _NB], sps[c % _NB])

    def gather(c):
        return [pltpu.async_copy(
            word_hbm.at[idx_v.at[b, pl.ds(c * _R, _R)]],
            wbuf.at[c % _NB, b], sgs[c % _NB]) for b in range(_B)]

    pls = {0: pos_load(0), 1: pos_load(1)}
    gs = {0: gather(0), 1: gather(1)}
    outs = {}
    for c in range(_NC):
        slot = c % _NB
        for g in gs[c]:
            g.wait()
        pls[c].wait()

        def add_body(r, _, c=c, slot=slot):
            for g in range(_D // 64):
                vals = [pos_v[slot, r, pl.ds((g * 4 + j) * 16, 16)]
                        for j in range(4)]
                for j in range(4):
                    for b in range(_B):
                        plsc.addupdate(
                            wbuf.at[slot, b, r, pl.ds((g * 4 + j) * 16, 16)],
                            vals[j])
            return 0

        lax.fori_loop(0, _R, add_body, 0)
        outs[c] = [pltpu.async_copy(
            wbuf.at[slot, b],
            out_hbm.at[b, pl.ds(pos_base + c * _R, _R)],
            sos[slot]) for b in range(_B)]
        if c + 2 < _NC:
            if c - 1 >= 0:
                for o in outs[c - 1]:
                    o.wait()
            pls[c + 2] = pos_load(c + 2)
            gs[c + 2] = gather(c + 2)
    for c in range(max(0, _NC - _NB), _NC):
        for o in outs[c]:
            o.wait()


@jax.jit
def _run(inputs, word_emb, pos_emb):
    mesh = plsc.VectorSubcoreMesh(core_axis_name="c", subcore_axis_name="s")
    k = functools.partial(
        pl.kernel,
        mesh=mesh,
        out_type=jax.ShapeDtypeStruct((_B, _S, _D), jnp.float32),
        scratch_types=[
            pltpu.VMEM((_B, _C), jnp.int32),
            pltpu.VMEM((_NB, _R, _D), jnp.float32),
            pltpu.VMEM((_NB, _B, _R, _D), jnp.float32),
            pltpu.SemaphoreType.DMA,
            pltpu.SemaphoreType.DMA,
            pltpu.SemaphoreType.DMA,
            pltpu.SemaphoreType.DMA,
            pltpu.SemaphoreType.DMA,
            pltpu.SemaphoreType.DMA,
            pltpu.SemaphoreType.DMA,
            pltpu.SemaphoreType.DMA,
            pltpu.SemaphoreType.DMA,
            pltpu.SemaphoreType.DMA,
        ],
    )(_emb_kernel)
    return k(inputs, word_emb, pos_emb)


def kernel(inputs, word_emb, pos_emb):
    return _run(inputs, word_emb, pos_emb)


# speedup vs baseline: 1.1312x; 1.0288x over previous
import functools

import jax
import jax.numpy as jnp
from jax import lax
from jax.experimental import pallas as pl
from jax.experimental.pallas import tpu as pltpu
from jax.experimental.pallas import tpu_sc as plsc

_B = 4
_S = 2048
_D = 512
_NW = 32
_C = _S // _NW
_R = 16
_NC = _C // _R


def _emb_kernel(idx_hbm, word_hbm, pos_hbm, out_hbm,
                idx_v, pos_v, wbuf, si, sp, sg0, sg1, so0, so1):
    wid = lax.axis_index("s") * 2 + lax.axis_index("c")
    pos_base = wid * _C
    idx_cps = [pltpu.async_copy(idx_hbm.at[b, pl.ds(pos_base, _C)],
                                idx_v.at[b], si) for b in range(_B)]
    pp = pltpu.async_copy(pos_hbm.at[pl.ds(pos_base, _C)], pos_v, sp)
    for c in idx_cps:
        c.wait()

    sgs, sos = (sg0, sg1), (so0, so1)

    def gather(chunk, buf):
        return [pltpu.async_copy(
            word_hbm.at[idx_v.at[b, pl.ds(chunk * _R, _R)]],
            wbuf.at[buf, b], sgs[buf]) for b in range(_B)]

    gs = [gather(0, 0), None]
    outs = [None, None]
    pp.wait()
    for c in range(_NC):
        cur, nxt = c % 2, (c + 1) % 2
        if c + 1 < _NC:
            if outs[nxt] is not None:
                for o in outs[nxt]:
                    o.wait()
            gs[nxt] = gather(c + 1, nxt)
        for g in gs[cur]:
            g.wait()

        def add_body(r, _, c=c, cur=cur):
            for g in range(_D // 64):
                vals = [pos_v[c * _R + r, pl.ds((g * 4 + j) * 16, 16)]
                        for j in range(4)]
                for j in range(4):
                    for b in range(_B):
                        plsc.addupdate(
                            wbuf.at[cur, b, r, pl.ds((g * 4 + j) * 16, 16)],
                            vals[j])
            return 0

        lax.fori_loop(0, _R, add_body, 0)
        outs[cur] = [pltpu.async_copy(
            wbuf.at[cur, b],
            out_hbm.at[b, pl.ds(pos_base + c * _R, _R)],
            sos[cur]) for b in range(_B)]
    for os in outs:
        for o in os:
            o.wait()


@jax.jit
def _run(inputs, word_emb, pos_emb):
    mesh = plsc.VectorSubcoreMesh(core_axis_name="c", subcore_axis_name="s")
    k = functools.partial(
        pl.kernel,
        mesh=mesh,
        out_type=jax.ShapeDtypeStruct((_B, _S, _D), jnp.float32),
        scratch_types=[
            pltpu.VMEM((_B, _C), jnp.int32),
            pltpu.VMEM((_C, _D), jnp.float32),
            pltpu.VMEM((2, _B, _R, _D), jnp.float32),
            pltpu.SemaphoreType.DMA,
            pltpu.SemaphoreType.DMA,
            pltpu.SemaphoreType.DMA,
            pltpu.SemaphoreType.DMA,
            pltpu.SemaphoreType.DMA,
            pltpu.SemaphoreType.DMA,
        ],
    )(_emb_kernel)
    return k(inputs, word_emb, pos_emb)


def kernel(inputs, word_emb, pos_emb):
    return _run(inputs, word_emb, pos_emb)
